# Initial kernel scaffold; baseline (speedup 1.0000x reference)
import functools

import jax
import jax.numpy as jnp
from jax import lax
from jax.experimental import pallas as pl
from jax.experimental.pallas import tpu as pltpu
from jax.experimental.pallas import tpu_sc as plsc

N = 10000
E = 320000
IN_F = 128
OUT_F = 64
H = 8
HD = H * OUT_F
G = 4
GW = HD // G
NEG_SLOPE = 0.2

NC = 2
NS = 16
NW = NC * NS
L = 16

CB = 128
NBLK = E // CB
TRIPS = -(-NBLK // NW)

ROWS_PER_TILE = 624
ROWS_LAST = N - 15 * ROWS_PER_TILE


def _prep_body(w3_ref, al_ref, ar_ref, ae_ref, emb_ref, w1_ref, b1_ref,
               w2_ref, b2_ref, wl_ref, wr_ref, ee_ref):
    w3 = w3_ref[...]
    al = al_ref[...]
    ar = ar_ref[...]
    dn = (((1,), (1,)), ((0,), (0,)))
    wl_ref[...] = lax.dot_general(al, w3, dn, preferred_element_type=jnp.float32)
    wr_ref[...] = lax.dot_general(ar, w3, dn, preferred_element_type=jnp.float32)
    emb = emb_ref[...]
    h1 = lax.dot_general(emb, w1_ref[...], (((1,), (1,)), ((), ())),
                         preferred_element_type=jnp.float32) + b1_ref[...]
    h1 = jnp.maximum(h1, 0.0)
    ep = lax.dot_general(h1, w2_ref[...], (((1,), (1,)), ((), ())),
                         preferred_element_type=jnp.float32) + b2_ref[...]
    ep = ep * ae_ref[...]
    col = lax.broadcasted_iota(jnp.int32, (HD, H), 0) // OUT_F
    hh = lax.broadcasted_iota(jnp.int32, (HD, H), 1)
    gm = (col == hh).astype(jnp.float32)
    ee_ref[...] = lax.dot_general(ep, gm, (((1,), (0,)), ((), ())),
                                  preferred_element_type=jnp.float32)


def _prep(W_fc, attn_l, attn_r, attn_e, edge_emb, w1, b1, w2, b2):
    w3 = W_fc.reshape(H, OUT_F, IN_F)
    al = attn_l.reshape(H, OUT_F)
    ar = attn_r.reshape(H, OUT_F)
    ae = attn_e.reshape(1, HD)
    b1r = b1.reshape(1, 2 * OUT_F)
    b2r = b2.reshape(1, HD)
    return pl.pallas_call(
        _prep_body,
        out_shape=(
            jax.ShapeDtypeStruct((H, IN_F), jnp.float32),
            jax.ShapeDtypeStruct((H, IN_F), jnp.float32),
            jax.ShapeDtypeStruct((H, H), jnp.float32),
        ),
    )(w3, al, ar, ae, edge_emb, w1, b1r, w2, b2r)


ROW_BLK = 1000


def _main_body(feat_ref, wf_ref, wres_ref, wl_ref, wr_ref,
               fs0_ref, fs1_ref, fs2_ref, fs3_ref, base_ref, elp_ref, erp_ref):
    f = feat_ref[...]
    dn = (((1,), (1,)), ((), ()))
    fs = lax.dot_general(f, wf_ref[...], dn, preferred_element_type=jnp.float32)
    rv = lax.dot_general(f, wres_ref[...], dn, preferred_element_type=jnp.float32)
    base_ref[...] = fs + rv
    for g, r in enumerate((fs0_ref, fs1_ref, fs2_ref, fs3_ref)):
        r[...] = fs[:, g * GW:(g + 1) * GW]
    el = lax.dot_general(f, wl_ref[...], dn, preferred_element_type=jnp.float32)
    er = lax.dot_general(f, wr_ref[...], dn, preferred_element_type=jnp.float32)
    elp_ref[...] = jnp.concatenate([el, el], axis=1)
    erp_ref[...] = jnp.concatenate([er, er], axis=1)


def _node_proj(feat, W_fc, res_fc_W, wl, wr):
    nb = N // ROW_BLK
    fs_sd = jax.ShapeDtypeStruct((N, GW), jnp.float32)
    return pl.pallas_call(
        _main_body,
        grid=(nb,),
        in_specs=[
            pl.BlockSpec((ROW_BLK, IN_F), lambda i: (i, 0)),
            pl.BlockSpec((HD, IN_F), lambda i: (0, 0)),
            pl.BlockSpec((HD, IN_F), lambda i: (0, 0)),
            pl.BlockSpec((H, IN_F), lambda i: (0, 0)),
            pl.BlockSpec((H, IN_F), lambda i: (0, 0)),
        ],
        out_specs=[
            pl.BlockSpec((ROW_BLK, GW), lambda i: (i, 0)),
            pl.BlockSpec((ROW_BLK, GW), lambda i: (i, 0)),
            pl.BlockSpec((ROW_BLK, GW), lambda i: (i, 0)),
            pl.BlockSpec((ROW_BLK, GW), lambda i: (i, 0)),
            pl.BlockSpec((ROW_BLK, HD), lambda i: (i, 0)),
            pl.BlockSpec((ROW_BLK, L), lambda i: (i, 0)),
            pl.BlockSpec((ROW_BLK, L), lambda i: (i, 0)),
        ],
        out_shape=(fs_sd, fs_sd, fs_sd, fs_sd,
                   jax.ShapeDtypeStruct((N, HD), jnp.float32),
                   jax.ShapeDtypeStruct((N, L), jnp.float32),
                   jax.ShapeDtypeStruct((N, L), jnp.float32)),
    )(feat, W_fc, res_fc_W, wl, wr)


def _sc_mesh():
    return plsc.VectorSubcoreMesh(core_axis_name="c", subcore_axis_name="s")


def _zero_shared(z_v, acc_sh, sid, width):
    zrows = z_v.shape[0]
    z16 = jnp.zeros((L,), jnp.float32)

    def zb(i, _):
        for k in range(width // L):
            z_v[i, pl.ds(k * L, L)] = z16
        return 0

    lax.fori_loop(0, zrows, zb, 0)
    base = sid * ROWS_PER_TILE

    @pl.when(sid < NS - 1)
    def _():
        pltpu.sync_copy(z_v.at[pl.ds(0, ROWS_PER_TILE)],
                        acc_sh.at[pl.ds(base, ROWS_PER_TILE)])

    @pl.when(sid == NS - 1)
    def _():
        pltpu.sync_copy(z_v, acc_sh.at[pl.ds(base, ROWS_LAST)])


def _copy_out_shared(acc_sh, out_h, cid, sid):
    base = sid * ROWS_PER_TILE

    @pl.when(sid < NS - 1)
    def _():
        pltpu.sync_copy(acc_sh.at[pl.ds(base, ROWS_PER_TILE)],
                        out_h.at[cid, pl.ds(base, ROWS_PER_TILE)])

    @pl.when(sid == NS - 1)
    def _():
        pltpu.sync_copy(acc_sh.at[pl.ds(base, ROWS_LAST)],
                        out_h.at[cid, pl.ds(base, ROWS_LAST)])


def _edge_logits_body(src_h, dst_h, et_h, elp_h, erp_h, ee_h, w_h,
                      ex_h, spart_h,
                      src_v, dst_v, et_v, elr_v, err_v, ex_v, ee_v, w_v,
                      z_v, s_acc, sem):
    cid = lax.axis_index("c")
    sid = lax.axis_index("s")
    wid = sid * NC + cid

    pltpu.sync_copy(ee_h, ee_v)
    pltpu.sync_copy(w_h, w_v)
    _zero_shared(z_v, s_acc, sid, L)
    plsc.subcore_barrier()

    iot = lax.iota(jnp.int32, L)
    hsel = jnp.bitwise_and(iot, 7)

    def blk(t, _):
        b = wid + t * NW

        @pl.when(b < NBLK)
        def _():
            base = b * CB
            pltpu.sync_copy(src_h.at[pl.ds(base, CB)], src_v)
            pltpu.sync_copy(dst_h.at[pl.ds(base, CB)], dst_v)
            pltpu.sync_copy(et_h.at[pl.ds(base, CB)], et_v)
            pltpu.async_copy(elp_h.at[src_v], elr_v, sem).wait()
            pltpu.async_copy(erp_h.at[dst_v], err_v, sem).wait()

            def edge(c, _):
                et = et_v[c]
                eidx = jnp.broadcast_to(et * H, (L,)) + hsel
                eev = plsc.load_gather(ee_v, [eidx])
                wv = plsc.load_gather(w_v, [jnp.broadcast_to(et, (L,))])
                lg = elr_v[c] + err_v[c] + eev
                lg = jnp.maximum(lg, NEG_SLOPE * lg) * wv
                ex_v[c] = jnp.exp(lg)
                return 0

            lax.fori_loop(0, CB, edge, 0)
            pltpu.sync_copy(ex_v, ex_h.at[pl.ds(base, CB)])
            pltpu.sync_copy(ex_v, s_acc.at[dst_v], add=True)
        return 0

    lax.fori_loop(0, TRIPS, blk, 0)
    plsc.subcore_barrier()
    _copy_out_shared(s_acc, spart_h, cid, sid)


def _edge_logits(src, dst, et, elp, erp, ee_flat, w_tab):
    k = pl.kernel(
        _edge_logits_body,
        out_type=(
            jax.ShapeDtypeStruct((E, L), jnp.float32),
            jax.ShapeDtypeStruct((NC, N, L), jnp.float32),
        ),
        mesh=_sc_mesh(),
        scratch_types=[
            pltpu.VMEM((CB,), jnp.int32),
            pltpu.VMEM((CB,), jnp.int32),
            pltpu.VMEM((CB,), jnp.int32),
            pltpu.VMEM((CB, L), jnp.float32),
            pltpu.VMEM((CB, L), jnp.float32),
            pltpu.VMEM((CB, L), jnp.float32),
            pltpu.VMEM((H * H,), jnp.float32),
            pltpu.VMEM((H,), jnp.float32),
            pltpu.VMEM((ROWS_LAST, L), jnp.float32),
            pltpu.VMEM_SHARED((N, L), jnp.float32),
            pltpu.SemaphoreType.DMA,
        ],
    )
    return k(src, dst, et, elp, erp, ee_flat, w_tab)


def _norm_body(ex_h, dst_h, spart_h, a_h,
               dst_v, ex_v, s0_v, s1_v, sem):
    cid = lax.axis_index("c")
    sid = lax.axis_index("s")
    wid = sid * NC + cid

    def blk(t, _):
        b = wid + t * NW

        @pl.when(b < NBLK)
        def _():
            base = b * CB
            pltpu.sync_copy(dst_h.at[pl.ds(base, CB)], dst_v)
            pltpu.sync_copy(ex_h.at[pl.ds(base, CB)], ex_v)
            pltpu.async_copy(spart_h.at[0].at[dst_v], s0_v, sem).wait()
            pltpu.async_copy(spart_h.at[1].at[dst_v], s1_v, sem).wait()

            def edge(c, _):
                ex_v[c] = ex_v[c] / (s0_v[c] + s1_v[c])
                return 0

            lax.fori_loop(0, CB, edge, 0)
            pltpu.sync_copy(ex_v, a_h.at[pl.ds(base, CB)])
        return 0

    lax.fori_loop(0, TRIPS, blk, 0)


def _normalize(ex, dst, spart):
    k = pl.kernel(
        _norm_body,
        out_type=jax.ShapeDtypeStruct((E, L), jnp.float32),
        mesh=_sc_mesh(),
        scratch_types=[
            pltpu.VMEM((CB,), jnp.int32),
            pltpu.VMEM((CB, L), jnp.float32),
            pltpu.VMEM((CB, L), jnp.float32),
            pltpu.VMEM((CB, L), jnp.float32),
            pltpu.SemaphoreType.DMA,
        ],
    )
    return k(ex, dst, spart)


def _agg_body(g, fs_h, src_h, dst_h, a_h, out_h,
              src_v, dst_v, a_v, fs_v, z_v, acc_sh, sem):
    cid = lax.axis_index("c")
    sid = lax.axis_index("s")
    wid = sid * NC + cid

    _zero_shared(z_v, acc_sh, sid, GW)
    plsc.subcore_barrier()

    def blk(t, _):
        b = wid + t * NW

        @pl.when(b < NBLK)
        def _():
            base = b * CB
            pltpu.sync_copy(src_h.at[pl.ds(base, CB)], src_v)
            pltpu.sync_copy(dst_h.at[pl.ds(base, CB)], dst_v)
            pltpu.sync_copy(a_h.at[pl.ds(base, CB)], a_v)
            pltpu.async_copy(fs_h.at[src_v], fs_v, sem).wait()

            def edge(c, _):
                v0 = jnp.broadcast_to(a_v[c, 2 * g], (L,))
                v1 = jnp.broadcast_to(a_v[c, 2 * g + 1], (L,))
                for kq in range(GW // L):
                    sc = v0 if kq < (GW // L) // 2 else v1
                    fs_v[c, pl.ds(kq * L, L)] = fs_v[c, pl.ds(kq * L, L)] * sc
                return 0

            lax.fori_loop(0, CB, edge, 0)
            pltpu.sync_copy(fs_v, acc_sh.at[dst_v], add=True)
        return 0

    lax.fori_loop(0, TRIPS, blk, 0)
    plsc.subcore_barrier()
    _copy_out_shared(acc_sh, out_h, cid, sid)


def _aggregate(g, fs_g, src, dst, a16):
    k = pl.kernel(
        functools.partial(_agg_body, g),
        out_type=jax.ShapeDtypeStruct((NC, N, GW), jnp.float32),
        mesh=_sc_mesh(),
        scratch_types=[
            pltpu.VMEM((CB,), jnp.int32),
            pltpu.VMEM((CB,), jnp.int32),
            pltpu.VMEM((CB, L), jnp.float32),
            pltpu.VMEM((CB, GW), jnp.float32),
            pltpu.VMEM((ROWS_LAST, GW), jnp.float32),
            pltpu.VMEM_SHARED((N, GW), jnp.float32),
            pltpu.SemaphoreType.DMA,
        ],
    )
    return k(fs_g, src, dst, a16)


def _combine_body(base_ref, o0_ref, o1_ref, o2_ref, o3_ref, out_ref):
    b = base_ref[...]
    outs = (o0_ref, o1_ref, o2_ref, o3_ref)
    for g in range(G):
        o = outs[g]
        out_ref[:, g * GW:(g + 1) * GW] = (
            b[:, g * GW:(g + 1) * GW] + o[0] + o[1])


def _combine(rst_base, parts):
    nb = N // ROW_BLK
    pspec = pl.BlockSpec((NC, ROW_BLK, GW), lambda i: (0, i, 0))
    return pl.pallas_call(
        _combine_body,
        grid=(nb,),
        in_specs=[pl.BlockSpec((ROW_BLK, HD), lambda i: (i, 0)),
                  pspec, pspec, pspec, pspec],
        out_specs=pl.BlockSpec((ROW_BLK, HD), lambda i: (i, 0)),
        out_shape=jax.ShapeDtypeStruct((N, HD), jnp.float32),
    )(rst_base, *parts)


def kernel(feat, edge_index, e_feat, W_fc, edge_emb, w_r,
           fc_e_W1, fc_e_b1, fc_e_W2, fc_e_b2,
           attn_l, attn_r, attn_e, res_fc_W):
    src = edge_index[0].astype(jnp.int32)
    dst = edge_index[1].astype(jnp.int32)
    et = e_feat.astype(jnp.int32)
    w_tab = w_r.reshape(H).astype(jnp.float32)

    wl, wr, ee_tab = _prep(W_fc, attn_l, attn_r, attn_e, edge_emb,
                           fc_e_W1, fc_e_b1, fc_e_W2, fc_e_b2)
    fs0, fs1, fs2, fs3, rst_base, elp, erp = _node_proj(
        feat, W_fc, res_fc_W, wl, wr)

    ex16, spart = _edge_logits(src, dst, et, elp, erp,
                               ee_tab.reshape(H * H), w_tab)
    a16 = _normalize(ex16, dst, spart)

    parts = [_aggregate(g, fs_g, src, dst, a16)
             for g, fs_g in enumerate((fs0, fs1, fs2, fs3))]

    rst = _combine(rst_base, parts)
    return rst.reshape(N, H, OUT_F), a16[:, :H]


# speedup vs baseline: 12.5316x; 12.5316x over previous
import functools

import jax
import jax.numpy as jnp
from jax import lax
from jax.experimental import pallas as pl
from jax.experimental.pallas import tpu as pltpu
from jax.experimental.pallas import tpu_sc as plsc

N = 10000
E = 320000
IN_F = 128
OUT_F = 64
H = 8
HD = H * OUT_F
G = 8
GW = HD // G
NEG_SLOPE = 0.2

NC = 2
NS = 16
NW = NC * NS
L = 16

CB = 128
NBLK = E // CB
TRIPS = -(-NBLK // NW)

ROWS_PER_TILE = 624
ROWS_LAST = N - 15 * ROWS_PER_TILE


def _prep_body(w3_ref, al_ref, ar_ref, ae_ref, emb_ref, w1_ref, b1_ref,
               w2_ref, b2_ref, wl_ref, wr_ref, ee_ref):
    w3 = w3_ref[...]
    al = al_ref[...]
    ar = ar_ref[...]
    dn = (((1,), (1,)), ((0,), (0,)))
    wl_ref[...] = lax.dot_general(al, w3, dn, preferred_element_type=jnp.float32)
    wr_ref[...] = lax.dot_general(ar, w3, dn, preferred_element_type=jnp.float32)
    emb = emb_ref[...]
    h1 = lax.dot_general(emb, w1_ref[...], (((1,), (1,)), ((), ())),
                         preferred_element_type=jnp.float32) + b1_ref[...]
    h1 = jnp.maximum(h1, 0.0)
    ep = lax.dot_general(h1, w2_ref[...], (((1,), (1,)), ((), ())),
                         preferred_element_type=jnp.float32) + b2_ref[...]
    ep = ep * ae_ref[...]
    col = lax.broadcasted_iota(jnp.int32, (HD, H), 0) // OUT_F
    hh = lax.broadcasted_iota(jnp.int32, (HD, H), 1)
    gm = (col == hh).astype(jnp.float32)
    ee = lax.dot_general(ep, gm, (((1,), (0,)), ((), ())),
                         preferred_element_type=jnp.float32)
    ee_ref[...] = jnp.concatenate([ee, ee], axis=1)


def _prep(W_fc, attn_l, attn_r, attn_e, edge_emb, w1, b1, w2, b2):
    w3 = W_fc.reshape(H, OUT_F, IN_F)
    al = attn_l.reshape(H, OUT_F)
    ar = attn_r.reshape(H, OUT_F)
    ae = attn_e.reshape(1, HD)
    b1r = b1.reshape(1, 2 * OUT_F)
    b2r = b2.reshape(1, HD)
    return pl.pallas_call(
        _prep_body,
        out_shape=(
            jax.ShapeDtypeStruct((H, IN_F), jnp.float32),
            jax.ShapeDtypeStruct((H, IN_F), jnp.float32),
            jax.ShapeDtypeStruct((H, L), jnp.float32),
        ),
    )(w3, al, ar, ae, edge_emb, w1, b1r, w2, b2r)


ROW_BLK = 1000


def _main_body(feat_ref, wf_ref, wres_ref, wl_ref, wr_ref,
               *out_refs):
    fs_refs = out_refs[:G]
    base_ref, elp_ref, erp_ref = out_refs[G:]
    f = feat_ref[...]
    dn = (((1,), (1,)), ((), ()))
    fs = lax.dot_general(f, wf_ref[...], dn, preferred_element_type=jnp.float32)
    rv = lax.dot_general(f, wres_ref[...], dn, preferred_element_type=jnp.float32)
    base_ref[...] = fs + rv
    for g, r in enumerate(fs_refs):
        r[...] = fs[:, g * GW:(g + 1) * GW]
    el = lax.dot_general(f, wl_ref[...], dn, preferred_element_type=jnp.float32)
    er = lax.dot_general(f, wr_ref[...], dn, preferred_element_type=jnp.float32)
    elp_ref[...] = jnp.concatenate([el, el], axis=1)
    erp_ref[...] = jnp.concatenate([er, er], axis=1)


def _node_proj(feat, W_fc, res_fc_W, wl, wr):
    nb = N // ROW_BLK
    fs_sd = jax.ShapeDtypeStruct((N, GW), jnp.float32)
    return pl.pallas_call(
        _main_body,
        grid=(nb,),
        in_specs=[
            pl.BlockSpec((ROW_BLK, IN_F), lambda i: (i, 0)),
            pl.BlockSpec((HD, IN_F), lambda i: (0, 0)),
            pl.BlockSpec((HD, IN_F), lambda i: (0, 0)),
            pl.BlockSpec((H, IN_F), lambda i: (0, 0)),
            pl.BlockSpec((H, IN_F), lambda i: (0, 0)),
        ],
        out_specs=[pl.BlockSpec((ROW_BLK, GW), lambda i: (i, 0))] * G + [
            pl.BlockSpec((ROW_BLK, HD), lambda i: (i, 0)),
            pl.BlockSpec((ROW_BLK, L), lambda i: (i, 0)),
            pl.BlockSpec((ROW_BLK, L), lambda i: (i, 0)),
        ],
        out_shape=tuple([fs_sd] * G) + (
                   jax.ShapeDtypeStruct((N, HD), jnp.float32),
                   jax.ShapeDtypeStruct((N, L), jnp.float32),
                   jax.ShapeDtypeStruct((N, L), jnp.float32)),
    )(feat, W_fc, res_fc_W, wl, wr)


def _sc_mesh():
    return plsc.VectorSubcoreMesh(core_axis_name="c", subcore_axis_name="s")


def _zero_shared(z_v, acc_sh, sid, width):
    zrows = z_v.shape[0]
    z16 = jnp.zeros((L,), jnp.float32)

    def zb(i, _):
        for k in range(width // L):
            z_v[i, pl.ds(k * L, L)] = z16
        return 0

    lax.fori_loop(0, zrows, zb, 0)
    base = sid * ROWS_PER_TILE

    @pl.when(sid < NS - 1)
    def _():
        pltpu.sync_copy(z_v.at[pl.ds(0, ROWS_PER_TILE)],
                        acc_sh.at[pl.ds(base, ROWS_PER_TILE)])

    @pl.when(sid == NS - 1)
    def _():
        pltpu.sync_copy(z_v, acc_sh.at[pl.ds(base, ROWS_LAST)])


def _copy_out_shared(acc_sh, out_h, cid, sid):
    base = sid * ROWS_PER_TILE

    @pl.when(sid < NS - 1)
    def _():
        pltpu.sync_copy(acc_sh.at[pl.ds(base, ROWS_PER_TILE)],
                        out_h.at[cid, pl.ds(base, ROWS_PER_TILE)])

    @pl.when(sid == NS - 1)
    def _():
        pltpu.sync_copy(acc_sh.at[pl.ds(base, ROWS_LAST)],
                        out_h.at[cid, pl.ds(base, ROWS_LAST)])


def _edge_logits_body(src_h, dst_h, et_h, elp_h, erp_h, ee_h, w_h,
                      ex_h, spart_h,
                      src_v, dst_v, et_v, elr_v, err_v, ex_v, ee_v, w_v,
                      z_v, s_acc, sem):
    cid = lax.axis_index("c")
    sid = lax.axis_index("s")
    wid = sid * NC + cid

    pltpu.sync_copy(ee_h, ee_v)
    pltpu.sync_copy(w_h, w_v)
    _zero_shared(z_v, s_acc, sid, L)
    plsc.subcore_barrier()

    def blk(t, _):
        b = wid + t * NW

        @pl.when(b < NBLK)
        def _():
            base = b * CB
            pltpu.sync_copy(src_h.at[pl.ds(base, CB)], src_v)
            pltpu.sync_copy(dst_h.at[pl.ds(base, CB)], dst_v)
            pltpu.sync_copy(et_h.at[pl.ds(base, CB)], et_v)
            pltpu.async_copy(elp_h.at[src_v], elr_v, sem).wait()
            pltpu.async_copy(erp_h.at[dst_v], err_v, sem).wait()

            def grp(j, _):
                etv = et_v[pl.ds(j * L, L)]
                for l in range(L):
                    c = j * L + l
                    et_s = etv[l]
                    lg = elr_v[c] + err_v[c] + ee_v[et_s]
                    lg = jnp.maximum(lg, NEG_SLOPE * lg) * w_v[et_s]
                    ex_v[c] = jnp.exp(lg)
                return 0

            lax.fori_loop(0, CB // L, grp, 0)
            pltpu.sync_copy(ex_v, ex_h.at[pl.ds(base, CB)])
            pltpu.sync_copy(ex_v, s_acc.at[dst_v], add=True)
        return 0

    lax.fori_loop(0, TRIPS, blk, 0)
    plsc.subcore_barrier()
    _copy_out_shared(s_acc, spart_h, cid, sid)


def _edge_logits(src, dst, et, elp, erp, ee16, w16):
    k = pl.kernel(
        _edge_logits_body,
        out_type=(
            jax.ShapeDtypeStruct((E, L), jnp.float32),
            jax.ShapeDtypeStruct((NC, N, L), jnp.float32),
        ),
        mesh=_sc_mesh(),
        compiler_params=pltpu.CompilerParams(use_tc_tiling_on_sc=False),
        scratch_types=[
            pltpu.VMEM((CB,), jnp.int32),
            pltpu.VMEM((CB,), jnp.int32),
            pltpu.VMEM((CB,), jnp.int32),
            pltpu.VMEM((CB, L), jnp.float32),
            pltpu.VMEM((CB, L), jnp.float32),
            pltpu.VMEM((CB, L), jnp.float32),
            pltpu.VMEM((H, L), jnp.float32),
            pltpu.VMEM((H, L), jnp.float32),
            pltpu.VMEM((ROWS_LAST, L), jnp.float32),
            pltpu.VMEM_SHARED((N, L), jnp.float32),
            pltpu.SemaphoreType.DMA,
        ],
    )
    return k(src, dst, et, elp, erp, ee16, w16)


def _norm_body(ex_h, dst_h, spart_h, a_h,
               dst_v, ex_v, s0_v, s1_v, sem):
    cid = lax.axis_index("c")
    sid = lax.axis_index("s")
    wid = sid * NC + cid

    def blk(t, _):
        b = wid + t * NW

        @pl.when(b < NBLK)
        def _():
            base = b * CB
            pltpu.sync_copy(dst_h.at[pl.ds(base, CB)], dst_v)
            pltpu.sync_copy(ex_h.at[pl.ds(base, CB)], ex_v)
            pltpu.async_copy(spart_h.at[0].at[dst_v], s0_v, sem).wait()
            pltpu.async_copy(spart_h.at[1].at[dst_v], s1_v, sem).wait()

            def edge(c, _):
                ex_v[c] = ex_v[c] / (s0_v[c] + s1_v[c])
                return 0

            lax.fori_loop(0, CB, edge, 0)
            pltpu.sync_copy(ex_v, a_h.at[pl.ds(base, CB)])
        return 0

    lax.fori_loop(0, TRIPS, blk, 0)


def _normalize(ex, dst, spart):
    k = pl.kernel(
        _norm_body,
        out_type=jax.ShapeDtypeStruct((E, L), jnp.float32),
        mesh=_sc_mesh(),
        compiler_params=pltpu.CompilerParams(use_tc_tiling_on_sc=False),
        scratch_types=[
            pltpu.VMEM((CB,), jnp.int32),
            pltpu.VMEM((CB, L), jnp.float32),
            pltpu.VMEM((CB, L), jnp.float32),
            pltpu.VMEM((CB, L), jnp.float32),
            pltpu.SemaphoreType.DMA,
        ],
    )
    return k(ex, dst, spart)


def _agg_body(g, fs_h, src_h, dst_h, a_h, out_h,
              src_v, dst_v, a_v, fs_v, z_v, acc_sh, sem):
    cid = lax.axis_index("c")
    sid = lax.axis_index("s")
    wid = sid * NC + cid

    _zero_shared(z_v, acc_sh, sid, GW)
    plsc.subcore_barrier()

    def blk(t, _):
        b = wid + t * NW

        @pl.when(b < NBLK)
        def _():
            base = b * CB
            pltpu.sync_copy(src_h.at[pl.ds(base, CB)], src_v)
            pltpu.sync_copy(dst_h.at[pl.ds(base, CB)], dst_v)
            pltpu.sync_copy(a_h.at[pl.ds(base, CB)], a_v)
            pltpu.async_copy(fs_h.at[src_v], fs_v, sem).wait()

            def edge(c, _):
                arow = a_v[c]
                v0 = jnp.broadcast_to(arow[g], (L,))
                for kq in range(GW // L):
                    fs_v[c, pl.ds(kq * L, L)] = fs_v[c, pl.ds(kq * L, L)] * v0
                return 0

            lax.fori_loop(0, CB, edge, 0)
            pltpu.sync_copy(fs_v, acc_sh.at[dst_v], add=True)
        return 0

    lax.fori_loop(0, TRIPS, blk, 0)
    plsc.subcore_barrier()
    _copy_out_shared(acc_sh, out_h, cid, sid)


def _aggregate(g, fs_g, src, dst, a16):
    k = pl.kernel(
        functools.partial(_agg_body, g),
        out_type=jax.ShapeDtypeStruct((NC, N, GW), jnp.float32),
        mesh=_sc_mesh(),
        compiler_params=pltpu.CompilerParams(use_tc_tiling_on_sc=False),
        scratch_types=[
            pltpu.VMEM((CB,), jnp.int32),
            pltpu.VMEM((CB,), jnp.int32),
            pltpu.VMEM((CB, L), jnp.float32),
            pltpu.VMEM((CB, GW), jnp.float32),
            pltpu.VMEM((ROWS_LAST, GW), jnp.float32),
            pltpu.VMEM_SHARED((N, GW), jnp.float32),
            pltpu.SemaphoreType.DMA,
        ],
    )
    return k(fs_g, src, dst, a16)


def _combine_body(base_ref, *refs):
    outs = refs[:G]
    out_ref = refs[G]
    b = base_ref[...]
    for g in range(G):
        o = outs[g]
        out_ref[:, g * GW:(g + 1) * GW] = (
            b[:, g * GW:(g + 1) * GW] + o[0] + o[1])


def _combine(rst_base, parts):
    nb = N // ROW_BLK
    pspec = pl.BlockSpec((NC, ROW_BLK, GW), lambda i: (0, i, 0))
    return pl.pallas_call(
        _combine_body,
        grid=(nb,),
        in_specs=[pl.BlockSpec((ROW_BLK, HD), lambda i: (i, 0))] + [pspec] * G,
        out_specs=pl.BlockSpec((ROW_BLK, HD), lambda i: (i, 0)),
        out_shape=jax.ShapeDtypeStruct((N, HD), jnp.float32),
    )(rst_base, *parts)


def kernel(feat, edge_index, e_feat, W_fc, edge_emb, w_r,
           fc_e_W1, fc_e_b1, fc_e_W2, fc_e_b2,
           attn_l, attn_r, attn_e, res_fc_W):
    src = edge_index[0].astype(jnp.int32)
    dst = edge_index[1].astype(jnp.int32)
    et = e_feat.astype(jnp.int32)
    w16 = jnp.broadcast_to(w_r.reshape(H, 1).astype(jnp.float32), (H, L))

    wl, wr, ee16 = _prep(W_fc, attn_l, attn_r, attn_e, edge_emb,
                         fc_e_W1, fc_e_b1, fc_e_W2, fc_e_b2)
    proj = _node_proj(feat, W_fc, res_fc_W, wl, wr)
    fs_slabs = proj[:G]
    rst_base, elp, erp = proj[G:]

    ex16, spart = _edge_logits(src, dst, et, elp, erp, ee16, w16)
    a16 = _normalize(ex16, dst, spart)

    parts = [_aggregate(g, fs_g, src, dst, a16)
             for g, fs_g in enumerate(fs_slabs)]

    rst = _combine(rst_base, parts)
    return rst.reshape(N, H, OUT_F), a16[:, :H]


# speedup vs baseline: 19.1770x; 1.5303x over previous
import functools

import jax
import jax.numpy as jnp
from jax import lax
from jax.experimental import pallas as pl
from jax.experimental.pallas import tpu as pltpu
from jax.experimental.pallas import tpu_sc as plsc

N = 10000
E = 320000
IN_F = 128
OUT_F = 64
H = 8
HD = H * OUT_F
G = 8
GW = HD // G
NEG_SLOPE = 0.2

NC = 2
NS = 16
NW = NC * NS
L = 16

CB = 128
SB = 4
CHUNK = SB * CB
NBLK = E // CHUNK
TRIPS = -(-NBLK // NW)

ROWS_PER_TILE = 624
ROWS_LAST = N - 15 * ROWS_PER_TILE


def _prep_body(w3_ref, al_ref, ar_ref, ae_ref, emb_ref, w1_ref, b1_ref,
               w2_ref, b2_ref, wl_ref, wr_ref, ee_ref):
    w3 = w3_ref[...]
    al = al_ref[...]
    ar = ar_ref[...]
    dn = (((1,), (1,)), ((0,), (0,)))
    wl_ref[...] = lax.dot_general(al, w3, dn, preferred_element_type=jnp.float32)
    wr_ref[...] = lax.dot_general(ar, w3, dn, preferred_element_type=jnp.float32)
    emb = emb_ref[...]
    h1 = lax.dot_general(emb, w1_ref[...], (((1,), (1,)), ((), ())),
                         preferred_element_type=jnp.float32) + b1_ref[...]
    h1 = jnp.maximum(h1, 0.0)
    ep = lax.dot_general(h1, w2_ref[...], (((1,), (1,)), ((), ())),
                         preferred_element_type=jnp.float32) + b2_ref[...]
    ep = ep * ae_ref[...]
    col = lax.broadcasted_iota(jnp.int32, (HD, H), 0) // OUT_F
    hh = lax.broadcasted_iota(jnp.int32, (HD, H), 1)
    gm = (col == hh).astype(jnp.float32)
    ee = lax.dot_general(ep, gm, (((1,), (0,)), ((), ())),
                         preferred_element_type=jnp.float32)
    ee_ref[...] = jnp.concatenate([ee, ee], axis=1)


def _prep(W_fc, attn_l, attn_r, attn_e, edge_emb, w1, b1, w2, b2):
    w3 = W_fc.reshape(H, OUT_F, IN_F)
    al = attn_l.reshape(H, OUT_F)
    ar = attn_r.reshape(H, OUT_F)
    ae = attn_e.reshape(1, HD)
    b1r = b1.reshape(1, 2 * OUT_F)
    b2r = b2.reshape(1, HD)
    return pl.pallas_call(
        _prep_body,
        out_shape=(
            jax.ShapeDtypeStruct((H, IN_F), jnp.float32),
            jax.ShapeDtypeStruct((H, IN_F), jnp.float32),
            jax.ShapeDtypeStruct((H, L), jnp.float32),
        ),
    )(w3, al, ar, ae, edge_emb, w1, b1r, w2, b2r)


ROW_BLK = 1000


def _main_body(feat_ref, wf_ref, wres_ref, wl_ref, wr_ref,
               *out_refs):
    fs_refs = out_refs[:G]
    base_ref, elp_ref, erp_ref = out_refs[G:]
    f = feat_ref[...]
    dn = (((1,), (1,)), ((), ()))
    fs = lax.dot_general(f, wf_ref[...], dn, preferred_element_type=jnp.float32)
    rv = lax.dot_general(f, wres_ref[...], dn, preferred_element_type=jnp.float32)
    base_ref[...] = fs + rv
    for g, r in enumerate(fs_refs):
        r[...] = fs[:, g * GW:(g + 1) * GW]
    el = lax.dot_general(f, wl_ref[...], dn, preferred_element_type=jnp.float32)
    er = lax.dot_general(f, wr_ref[...], dn, preferred_element_type=jnp.float32)
    elp_ref[...] = jnp.concatenate([el, el], axis=1)
    erp_ref[...] = jnp.concatenate([er, er], axis=1)


def _node_proj(feat, W_fc, res_fc_W, wl, wr):
    nb = N // ROW_BLK
    fs_sd = jax.ShapeDtypeStruct((N, GW), jnp.float32)
    return pl.pallas_call(
        _main_body,
        grid=(nb,),
        in_specs=[
            pl.BlockSpec((ROW_BLK, IN_F), lambda i: (i, 0)),
            pl.BlockSpec((HD, IN_F), lambda i: (0, 0)),
            pl.BlockSpec((HD, IN_F), lambda i: (0, 0)),
            pl.BlockSpec((H, IN_F), lambda i: (0, 0)),
            pl.BlockSpec((H, IN_F), lambda i: (0, 0)),
        ],
        out_specs=[pl.BlockSpec((ROW_BLK, GW), lambda i: (i, 0))] * G + [
            pl.BlockSpec((ROW_BLK, HD), lambda i: (i, 0)),
            pl.BlockSpec((ROW_BLK, L), lambda i: (i, 0)),
            pl.BlockSpec((ROW_BLK, L), lambda i: (i, 0)),
        ],
        out_shape=tuple([fs_sd] * G) + (
                   jax.ShapeDtypeStruct((N, HD), jnp.float32),
                   jax.ShapeDtypeStruct((N, L), jnp.float32),
                   jax.ShapeDtypeStruct((N, L), jnp.float32)),
    )(feat, W_fc, res_fc_W, wl, wr)


def _sc_mesh():
    return plsc.VectorSubcoreMesh(core_axis_name="c", subcore_axis_name="s")


def _zero_shared(z_v, acc_sh, sid, width):
    zrows = z_v.shape[0]
    z16 = jnp.zeros((L,), jnp.float32)

    def zb(i, _):
        for k in range(width // L):
            z_v[i, pl.ds(k * L, L)] = z16
        return 0

    lax.fori_loop(0, zrows, zb, 0)
    base = sid * ROWS_PER_TILE

    @pl.when(sid < NS - 1)
    def _():
        pltpu.sync_copy(z_v.at[pl.ds(0, ROWS_PER_TILE)],
                        acc_sh.at[pl.ds(base, ROWS_PER_TILE)])

    @pl.when(sid == NS - 1)
    def _():
        pltpu.sync_copy(z_v, acc_sh.at[pl.ds(base, ROWS_LAST)])


def _copy_out_shared(acc_sh, out_h, cid, sid):
    base = sid * ROWS_PER_TILE

    @pl.when(sid < NS - 1)
    def _():
        pltpu.sync_copy(acc_sh.at[pl.ds(base, ROWS_PER_TILE)],
                        out_h.at[cid, pl.ds(base, ROWS_PER_TILE)])

    @pl.when(sid == NS - 1)
    def _():
        pltpu.sync_copy(acc_sh.at[pl.ds(base, ROWS_LAST)],
                        out_h.at[cid, pl.ds(base, ROWS_LAST)])


def _edge_logits_body(src_h, dst_h, et_h, elp_h, erp_h, ee_h, w_h,
                      ex_h, spart_h,
                      src_v, dst_v, et_v, elr_v, err_v, ex_v, ee_v, w_v,
                      z_v, s_acc, sem):
    cid = lax.axis_index("c")
    sid = lax.axis_index("s")
    wid = sid * NC + cid

    pltpu.sync_copy(ee_h, ee_v)
    pltpu.sync_copy(w_h, w_v)
    _zero_shared(z_v, s_acc, sid, L)
    plsc.subcore_barrier()

    def blk(t, _):
        b = wid + t * NW

        @pl.when(b < NBLK)
        def _():
            base = b * CHUNK
            pltpu.sync_copy(src_h.at[pl.ds(b * SB, SB)], src_v)
            pltpu.sync_copy(dst_h.at[pl.ds(b * SB, SB)], dst_v)
            pltpu.sync_copy(et_h.at[pl.ds(b * SB, SB)], et_v)
            cps = []
            for j in range(SB):
                cps.append(pltpu.async_copy(
                    elp_h.at[src_v.at[j]], elr_v.at[pl.ds(j * CB, CB)], sem))
                cps.append(pltpu.async_copy(
                    erp_h.at[dst_v.at[j]], err_v.at[pl.ds(j * CB, CB)], sem))
            for cp in cps:
                cp.wait()

            for j in range(SB):
                def grp(g2, _, j=j):
                    etv = et_v[j, pl.ds(g2 * L, L)]
                    for l in range(L):
                        c = j * CB + g2 * L + l
                        et_s = etv[l]
                        lg = elr_v[c] + err_v[c] + ee_v[et_s]
                        lg = jnp.maximum(lg, NEG_SLOPE * lg) * w_v[et_s]
                        ex_v[c] = jnp.exp(lg)
                    return 0

                lax.fori_loop(0, CB // L, grp, 0)
            pltpu.sync_copy(ex_v, ex_h.at[pl.ds(base, CHUNK)])
            for j in range(SB):
                pltpu.sync_copy(ex_v.at[pl.ds(j * CB, CB)],
                                s_acc.at[dst_v.at[j]], add=True)
        return 0

    lax.fori_loop(0, TRIPS, blk, 0)
    plsc.subcore_barrier()
    _copy_out_shared(s_acc, spart_h, cid, sid)


def _edge_logits(src, dst, et, elp, erp, ee16, w16):
    k = pl.kernel(
        _edge_logits_body,
        out_type=(
            jax.ShapeDtypeStruct((E, L), jnp.float32),
            jax.ShapeDtypeStruct((NC, N, L), jnp.float32),
        ),
        mesh=_sc_mesh(),
        compiler_params=pltpu.CompilerParams(use_tc_tiling_on_sc=False),
        scratch_types=[
            pltpu.VMEM((SB, CB), jnp.int32),
            pltpu.VMEM((SB, CB), jnp.int32),
            pltpu.VMEM((SB, CB), jnp.int32),
            pltpu.VMEM((CHUNK, L), jnp.float32),
            pltpu.VMEM((CHUNK, L), jnp.float32),
            pltpu.VMEM((CHUNK, L), jnp.float32),
            pltpu.VMEM((H, L), jnp.float32),
            pltpu.VMEM((H, L), jnp.float32),
            pltpu.VMEM((ROWS_LAST, L), jnp.float32),
            pltpu.VMEM_SHARED((N, L), jnp.float32),
            pltpu.SemaphoreType.DMA,
        ],
    )
    return k(src, dst, et, elp, erp, ee16, w16)


def _norm_body(ex_h, dst_h, spart_h, a_h,
               dst_v, ex_v, s0_v, s1_v, sem):
    cid = lax.axis_index("c")
    sid = lax.axis_index("s")
    wid = sid * NC + cid

    def blk(t, _):
        b = wid + t * NW

        @pl.when(b < NBLK)
        def _():
            base = b * CHUNK
            pltpu.sync_copy(dst_h.at[pl.ds(b * SB, SB)], dst_v)
            pltpu.sync_copy(ex_h.at[pl.ds(base, CHUNK)], ex_v)
            cps = []
            for j in range(SB):
                cps.append(pltpu.async_copy(
                    spart_h.at[0].at[dst_v.at[j]],
                    s0_v.at[pl.ds(j * CB, CB)], sem))
                cps.append(pltpu.async_copy(
                    spart_h.at[1].at[dst_v.at[j]],
                    s1_v.at[pl.ds(j * CB, CB)], sem))
            for cp in cps:
                cp.wait()

            def edge(c, _):
                ex_v[c] = ex_v[c] / (s0_v[c] + s1_v[c])
                return 0

            lax.fori_loop(0, CHUNK, edge, 0, unroll=4)
            pltpu.sync_copy(ex_v, a_h.at[pl.ds(base, CHUNK)])
        return 0

    lax.fori_loop(0, TRIPS, blk, 0)


def _normalize(ex, dst, spart):
    k = pl.kernel(
        _norm_body,
        out_type=jax.ShapeDtypeStruct((E, L), jnp.float32),
        mesh=_sc_mesh(),
        compiler_params=pltpu.CompilerParams(use_tc_tiling_on_sc=False),
        scratch_types=[
            pltpu.VMEM((SB, CB), jnp.int32),
            pltpu.VMEM((CHUNK, L), jnp.float32),
            pltpu.VMEM((CHUNK, L), jnp.float32),
            pltpu.VMEM((CHUNK, L), jnp.float32),
            pltpu.SemaphoreType.DMA,
        ],
    )
    return k(ex, dst, spart)


def _agg_body(g, fs_h, src_h, dst_h, a_h, out_h,
              src_v, dst_v, a_v, fs_v, z_v, acc_sh, sem):
    cid = lax.axis_index("c")
    sid = lax.axis_index("s")
    wid = sid * NC + cid

    _zero_shared(z_v, acc_sh, sid, GW)
    plsc.subcore_barrier()

    def blk(t, _):
        b = wid + t * NW

        @pl.when(b < NBLK)
        def _():
            base = b * CHUNK
            pltpu.sync_copy(src_h.at[pl.ds(b * SB, SB)], src_v)
            pltpu.sync_copy(dst_h.at[pl.ds(b * SB, SB)], dst_v)
            pltpu.sync_copy(a_h.at[pl.ds(base, CHUNK)], a_v)
            cps = [pltpu.async_copy(fs_h.at[src_v.at[j]],
                                    fs_v.at[pl.ds(j * CB, CB)], sem)
                   for j in range(SB)]
            for cp in cps:
                cp.wait()

            def edge(c, _):
                arow = a_v[c]
                v0 = jnp.broadcast_to(arow[g], (L,))
                for kq in range(GW // L):
                    fs_v[c, pl.ds(kq * L, L)] = fs_v[c, pl.ds(kq * L, L)] * v0
                return 0

            lax.fori_loop(0, CHUNK, edge, 0, unroll=2)
            for j in range(SB):
                pltpu.sync_copy(fs_v.at[pl.ds(j * CB, CB)],
                                acc_sh.at[dst_v.at[j]], add=True)
        return 0

    lax.fori_loop(0, TRIPS, blk, 0)
    plsc.subcore_barrier()
    _copy_out_shared(acc_sh, out_h, cid, sid)


def _aggregate(g, fs_g, src, dst, a16):
    k = pl.kernel(
        functools.partial(_agg_body, g),
        out_type=jax.ShapeDtypeStruct((NC, N, GW), jnp.float32),
        mesh=_sc_mesh(),
        compiler_params=pltpu.CompilerParams(use_tc_tiling_on_sc=False),
        scratch_types=[
            pltpu.VMEM((SB, CB), jnp.int32),
            pltpu.VMEM((SB, CB), jnp.int32),
            pltpu.VMEM((CHUNK, L), jnp.float32),
            pltpu.VMEM((CHUNK, GW), jnp.float32),
            pltpu.VMEM((ROWS_LAST, GW), jnp.float32),
            pltpu.VMEM_SHARED((N, GW), jnp.float32),
            pltpu.SemaphoreType.DMA,
        ],
    )
    return k(fs_g, src, dst, a16)


def _combine_body(base_ref, *refs):
    outs = refs[:G]
    out_ref = refs[G]
    b = base_ref[...]
    for g in range(G):
        o = outs[g]
        out_ref[:, g * GW:(g + 1) * GW] = (
            b[:, g * GW:(g + 1) * GW] + o[0] + o[1])


def _combine(rst_base, parts):
    nb = N // ROW_BLK
    pspec = pl.BlockSpec((NC, ROW_BLK, GW), lambda i: (0, i, 0))
    return pl.pallas_call(
        _combine_body,
        grid=(nb,),
        in_specs=[pl.BlockSpec((ROW_BLK, HD), lambda i: (i, 0))] + [pspec] * G,
        out_specs=pl.BlockSpec((ROW_BLK, HD), lambda i: (i, 0)),
        out_shape=jax.ShapeDtypeStruct((N, HD), jnp.float32),
    )(rst_base, *parts)


def kernel(feat, edge_index, e_feat, W_fc, edge_emb, w_r,
           fc_e_W1, fc_e_b1, fc_e_W2, fc_e_b2,
           attn_l, attn_r, attn_e, res_fc_W):
    src = edge_index[0].astype(jnp.int32).reshape(E // CB, CB)
    dst = edge_index[1].astype(jnp.int32).reshape(E // CB, CB)
    et = e_feat.astype(jnp.int32).reshape(E // CB, CB)
    w16 = jnp.broadcast_to(w_r.reshape(H, 1).astype(jnp.float32), (H, L))

    wl, wr, ee16 = _prep(W_fc, attn_l, attn_r, attn_e, edge_emb,
                         fc_e_W1, fc_e_b1, fc_e_W2, fc_e_b2)
    proj = _node_proj(feat, W_fc, res_fc_W, wl, wr)
    fs_slabs = proj[:G]
    rst_base, elp, erp = proj[G:]

    ex16, spart = _edge_logits(src, dst, et, elp, erp, ee16, w16)
    a16 = _normalize(ex16, dst, spart)

    parts = [_aggregate(g, fs_g, src, dst, a16)
             for g, fs_g in enumerate(fs_slabs)]

    rst = _combine(rst_base, parts)
    return rst.reshape(N, H, OUT_F), a16[:, :H]


# speedup vs baseline: 24.6718x; 1.2865x over previous
import functools

import jax
import jax.numpy as jnp
from jax import lax
from jax.experimental import pallas as pl
from jax.experimental.pallas import tpu as pltpu
from jax.experimental.pallas import tpu_sc as plsc

N = 10000
E = 320000
IN_F = 128
OUT_F = 64
H = 8
HD = H * OUT_F
G = 8
GW = HD // G
NEG_SLOPE = 0.2

NC = 2
NS = 16
NW = NC * NS
L = 16

CB = 128
SB = 4
CHUNK = SB * CB
NBLK = E // CHUNK
TRIPS = -(-NBLK // NW)
SB2 = 2
CHUNK2 = SB2 * CB
NBLK2 = E // CHUNK2
TRIPS2 = -(-NBLK2 // NW)

ROWS_PER_TILE = 624
ROWS_LAST = N - 15 * ROWS_PER_TILE


def _prep_body(w3_ref, al_ref, ar_ref, ae_ref, emb_ref, w1_ref, b1_ref,
               w2_ref, b2_ref, wl_ref, wr_ref, ee_ref):
    w3 = w3_ref[...]
    al = al_ref[...]
    ar = ar_ref[...]
    dn = (((1,), (1,)), ((0,), (0,)))
    wl_ref[...] = lax.dot_general(al, w3, dn, preferred_element_type=jnp.float32)
    wr_ref[...] = lax.dot_general(ar, w3, dn, preferred_element_type=jnp.float32)
    emb = emb_ref[...]
    h1 = lax.dot_general(emb, w1_ref[...], (((1,), (1,)), ((), ())),
                         preferred_element_type=jnp.float32) + b1_ref[...]
    h1 = jnp.maximum(h1, 0.0)
    ep = lax.dot_general(h1, w2_ref[...], (((1,), (1,)), ((), ())),
                         preferred_element_type=jnp.float32) + b2_ref[...]
    ep = ep * ae_ref[...]
    col = lax.broadcasted_iota(jnp.int32, (HD, H), 0) // OUT_F
    hh = lax.broadcasted_iota(jnp.int32, (HD, H), 1)
    gm = (col == hh).astype(jnp.float32)
    ee = lax.dot_general(ep, gm, (((1,), (0,)), ((), ())),
                         preferred_element_type=jnp.float32)
    ee_ref[...] = jnp.concatenate([ee, ee], axis=1)


def _prep(W_fc, attn_l, attn_r, attn_e, edge_emb, w1, b1, w2, b2):
    w3 = W_fc.reshape(H, OUT_F, IN_F)
    al = attn_l.reshape(H, OUT_F)
    ar = attn_r.reshape(H, OUT_F)
    ae = attn_e.reshape(1, HD)
    b1r = b1.reshape(1, 2 * OUT_F)
    b2r = b2.reshape(1, HD)
    return pl.pallas_call(
        _prep_body,
        out_shape=(
            jax.ShapeDtypeStruct((H, IN_F), jnp.float32),
            jax.ShapeDtypeStruct((H, IN_F), jnp.float32),
            jax.ShapeDtypeStruct((H, L), jnp.float32),
        ),
    )(w3, al, ar, ae, edge_emb, w1, b1r, w2, b2r)


ROW_BLK = 1000


def _main_body(feat_ref, wf_ref, wres_ref, wl_ref, wr_ref,
               *out_refs):
    fs_refs = out_refs[:G]
    base_ref, elp_ref, erp_ref = out_refs[G:]
    f = feat_ref[...]
    dn = (((1,), (1,)), ((), ()))
    fs = lax.dot_general(f, wf_ref[...], dn, preferred_element_type=jnp.float32)
    rv = lax.dot_general(f, wres_ref[...], dn, preferred_element_type=jnp.float32)
    base_ref[...] = fs + rv
    for g, r in enumerate(fs_refs):
        r[...] = fs[:, g * GW:(g + 1) * GW]
    el = lax.dot_general(f, wl_ref[...], dn, preferred_element_type=jnp.float32)
    er = lax.dot_general(f, wr_ref[...], dn, preferred_element_type=jnp.float32)
    elp_ref[...] = jnp.concatenate([el, el], axis=1)
    erp_ref[...] = jnp.concatenate([er, er], axis=1)


def _node_proj(feat, W_fc, res_fc_W, wl, wr):
    nb = N // ROW_BLK
    fs_sd = jax.ShapeDtypeStruct((N, GW), jnp.float32)
    return pl.pallas_call(
        _main_body,
        grid=(nb,),
        in_specs=[
            pl.BlockSpec((ROW_BLK, IN_F), lambda i: (i, 0)),
            pl.BlockSpec((HD, IN_F), lambda i: (0, 0)),
            pl.BlockSpec((HD, IN_F), lambda i: (0, 0)),
            pl.BlockSpec((H, IN_F), lambda i: (0, 0)),
            pl.BlockSpec((H, IN_F), lambda i: (0, 0)),
        ],
        out_specs=[pl.BlockSpec((ROW_BLK, GW), lambda i: (i, 0))] * G + [
            pl.BlockSpec((ROW_BLK, HD), lambda i: (i, 0)),
            pl.BlockSpec((ROW_BLK, L), lambda i: (i, 0)),
            pl.BlockSpec((ROW_BLK, L), lambda i: (i, 0)),
        ],
        out_shape=tuple([fs_sd] * G) + (
                   jax.ShapeDtypeStruct((N, HD), jnp.float32),
                   jax.ShapeDtypeStruct((N, L), jnp.float32),
                   jax.ShapeDtypeStruct((N, L), jnp.float32)),
    )(feat, W_fc, res_fc_W, wl, wr)


def _sc_mesh():
    return plsc.VectorSubcoreMesh(core_axis_name="c", subcore_axis_name="s")


def _db_loop(wid, issue_idx, drain_idx, fire_g, drain_g, body,
             nblk=NBLK, trips=TRIPS):
    def at(t):
        return wid + t * NW

    b0 = at(0)

    @pl.when(b0 < nblk)
    def _():
        issue_idx(b0, 0)
        drain_idx(b0, 0)
        fire_g(b0, 0)

    b1 = at(1)

    @pl.when(b1 < nblk)
    def _():
        issue_idx(b1, 1)

    def super_(tt, _):
        for p in (0, 1):
            t = tt * 2 + p
            b = at(t)
            bn = at(t + 1)
            bn2 = at(t + 2)
            np_ = 1 - p

            @pl.when(bn < nblk)
            def _(bn=bn, np_=np_):
                drain_idx(bn, np_)
                fire_g(bn, np_)

            @pl.when(b < nblk)
            def _(b=b, p=p):
                drain_g(b, p)
                body(b, p)

            @pl.when(bn2 < nblk)
            def _(bn2=bn2, p=p):
                issue_idx(bn2, p)
        return 0

    lax.fori_loop(0, (trips + 1) // 2, super_, 0)


def _zero_shared(z_v, acc_sh, sid, width):
    zrows = z_v.shape[0]
    z16 = jnp.zeros((L,), jnp.float32)

    def zb(i, _):
        for k in range(width // L):
            z_v[i, pl.ds(k * L, L)] = z16
        return 0

    lax.fori_loop(0, zrows, zb, 0)
    base = sid * ROWS_PER_TILE
    off = 0
    while off < ROWS_LAST:
        n_a = min(zrows, ROWS_PER_TILE - off) if off < ROWS_PER_TILE else 0
        n_b = min(zrows, ROWS_LAST - off)
        if n_a == n_b:
            pltpu.sync_copy(z_v.at[pl.ds(0, n_a)],
                            acc_sh.at[pl.ds(base + off, n_a)])
        else:
            if n_a > 0:
                @pl.when(sid < NS - 1)
                def _(n_a=n_a, off=off):
                    pltpu.sync_copy(z_v.at[pl.ds(0, n_a)],
                                    acc_sh.at[pl.ds(base + off, n_a)])

            @pl.when(sid == NS - 1)
            def _(n_b=n_b, off=off):
                pltpu.sync_copy(z_v.at[pl.ds(0, n_b)],
                                acc_sh.at[pl.ds(base + off, n_b)])
        off += zrows


def _copy_out_shared(acc_sh, out_h, cid, sid):
    base = sid * ROWS_PER_TILE

    @pl.when(sid < NS - 1)
    def _():
        pltpu.sync_copy(acc_sh.at[pl.ds(base, ROWS_PER_TILE)],
                        out_h.at[cid, pl.ds(base, ROWS_PER_TILE)])

    @pl.when(sid == NS - 1)
    def _():
        pltpu.sync_copy(acc_sh.at[pl.ds(base, ROWS_LAST)],
                        out_h.at[cid, pl.ds(base, ROWS_LAST)])


def _edge_logits_body(src_h, dst_h, et_h, elp_h, erp_h, ee_h, w_h,
                      ex_h, spart_h,
                      src_v, dst_v, et_v, elr_v, err_v, ex_v, ee_v, w_v,
                      z_v, s_acc, semi0, semi1, semg0, semg1):
    cid = lax.axis_index("c")
    sid = lax.axis_index("s")
    wid = sid * NC + cid
    semi = (semi0, semi1)
    semg = (semg0, semg1)

    pltpu.sync_copy(ee_h, ee_v)
    pltpu.sync_copy(w_h, w_v)
    _zero_shared(z_v, s_acc, sid, L)
    plsc.subcore_barrier()

    def idx_cps(b, p):
        s = pl.ds(b * SB, SB)
        return [(src_h.at[s], src_v.at[p]), (dst_h.at[s], dst_v.at[p]),
                (et_h.at[s], et_v.at[p])]

    def g_cps(b, p):
        out = []
        for j in range(SB):
            d = pl.ds(j * CB, CB)
            out.append((elp_h.at[src_v.at[p].at[j]], elr_v.at[p].at[d]))
            out.append((erp_h.at[dst_v.at[p].at[j]], err_v.at[p].at[d]))
        return out

    def issue_idx(b, p):
        for s, d in idx_cps(b, p):
            pltpu.async_copy(s, d, semi[p])

    def drain_idx(b, p):
        for s, d in idx_cps(b, p):
            pltpu.make_async_copy(s, d, semi[p]).wait()

    def fire_g(b, p):
        for s, d in g_cps(b, p):
            pltpu.async_copy(s, d, semg[p])

    def drain_g(b, p):
        for s, d in g_cps(b, p):
            pltpu.make_async_copy(s, d, semg[p]).wait()

    def body(b, p):
        base = b * CHUNK
        for j in range(SB):
            def grp(g2, _, j=j):
                etv = et_v[p, j, pl.ds(g2 * L, L)]
                for l in range(L):
                    c = j * CB + g2 * L + l
                    et_s = etv[l]
                    lg = elr_v[p, c] + err_v[p, c] + ee_v[et_s]
                    lg = jnp.maximum(lg, NEG_SLOPE * lg) * w_v[et_s]
                    ex_v[p, c] = jnp.exp(lg)
                return 0

            lax.fori_loop(0, CB // L, grp, 0)
        pltpu.sync_copy(ex_v.at[p], ex_h.at[pl.ds(base, CHUNK)])
        for j in range(SB):
            pltpu.sync_copy(ex_v.at[p].at[pl.ds(j * CB, CB)],
                            s_acc.at[dst_v.at[p].at[j]], add=True)

    _db_loop(wid, issue_idx, drain_idx, fire_g, drain_g, body)
    plsc.subcore_barrier()
    _copy_out_shared(s_acc, spart_h, cid, sid)


def _edge_logits(src, dst, et, elp, erp, ee16, w16):
    k = pl.kernel(
        _edge_logits_body,
        out_type=(
            jax.ShapeDtypeStruct((E, L), jnp.float32),
            jax.ShapeDtypeStruct((NC, N, L), jnp.float32),
        ),
        mesh=_sc_mesh(),
        compiler_params=pltpu.CompilerParams(use_tc_tiling_on_sc=False),
        scratch_types=[
            pltpu.VMEM((2, SB, CB), jnp.int32),
            pltpu.VMEM((2, SB, CB), jnp.int32),
            pltpu.VMEM((2, SB, CB), jnp.int32),
            pltpu.VMEM((2, CHUNK, L), jnp.float32),
            pltpu.VMEM((2, CHUNK, L), jnp.float32),
            pltpu.VMEM((2, CHUNK, L), jnp.float32),
            pltpu.VMEM((H, L), jnp.float32),
            pltpu.VMEM((H, L), jnp.float32),
            pltpu.VMEM((ROWS_LAST, L), jnp.float32),
            pltpu.VMEM_SHARED((N, L), jnp.float32),
            pltpu.SemaphoreType.DMA,
            pltpu.SemaphoreType.DMA,
            pltpu.SemaphoreType.DMA,
            pltpu.SemaphoreType.DMA,
        ],
    )
    return k(src, dst, et, elp, erp, ee16, w16)


def _norm_body(ex_h, dst_h, spart_h, a_h,
               dst_v, ex_v, s0_v, s1_v, semi0, semi1, semg0, semg1):
    cid = lax.axis_index("c")
    sid = lax.axis_index("s")
    wid = sid * NC + cid
    semi = (semi0, semi1)
    semg = (semg0, semg1)

    def idx_cps(b, p):
        return [(dst_h.at[pl.ds(b * SB, SB)], dst_v.at[p]),
                (ex_h.at[pl.ds(b * CHUNK, CHUNK)], ex_v.at[p])]

    def g_cps(b, p):
        out = []
        for j in range(SB):
            d = pl.ds(j * CB, CB)
            out.append((spart_h.at[0].at[dst_v.at[p].at[j]], s0_v.at[p].at[d]))
            out.append((spart_h.at[1].at[dst_v.at[p].at[j]], s1_v.at[p].at[d]))
        return out

    def issue_idx(b, p):
        for s, d in idx_cps(b, p):
            pltpu.async_copy(s, d, semi[p])

    def drain_idx(b, p):
        for s, d in idx_cps(b, p):
            pltpu.make_async_copy(s, d, semi[p]).wait()

    def fire_g(b, p):
        for s, d in g_cps(b, p):
            pltpu.async_copy(s, d, semg[p])

    def drain_g(b, p):
        for s, d in g_cps(b, p):
            pltpu.make_async_copy(s, d, semg[p]).wait()

    def body(b, p):
        def edge(c, _):
            ex_v[p, c] = ex_v[p, c] / (s0_v[p, c] + s1_v[p, c])
            return 0

        lax.fori_loop(0, CHUNK, edge, 0, unroll=4)
        pltpu.sync_copy(ex_v.at[p], a_h.at[pl.ds(b * CHUNK, CHUNK)])

    _db_loop(wid, issue_idx, drain_idx, fire_g, drain_g, body)


def _normalize(ex, dst, spart):
    k = pl.kernel(
        _norm_body,
        out_type=jax.ShapeDtypeStruct((E, L), jnp.float32),
        mesh=_sc_mesh(),
        compiler_params=pltpu.CompilerParams(use_tc_tiling_on_sc=False),
        scratch_types=[
            pltpu.VMEM((2, SB, CB), jnp.int32),
            pltpu.VMEM((2, CHUNK, L), jnp.float32),
            pltpu.VMEM((2, CHUNK, L), jnp.float32),
            pltpu.VMEM((2, CHUNK, L), jnp.float32),
            pltpu.SemaphoreType.DMA,
            pltpu.SemaphoreType.DMA,
            pltpu.SemaphoreType.DMA,
            pltpu.SemaphoreType.DMA,
        ],
    )
    return k(ex, dst, spart)


def _agg_body(g, fs_h, src_h, dst_h, a_h, out_h,
              src_v, dst_v, a_v, fs_v, z_v, acc_sh,
              semi0, semi1, semg0, semg1):
    cid = lax.axis_index("c")
    sid = lax.axis_index("s")
    wid = sid * NC + cid
    semi = (semi0, semi1)
    semg = (semg0, semg1)

    _zero_shared(z_v, acc_sh, sid, GW)
    plsc.subcore_barrier()

    def idx_cps(b, p):
        s = pl.ds(b * SB2, SB2)
        return [(src_h.at[s], src_v.at[p]), (dst_h.at[s], dst_v.at[p]),
                (a_h.at[pl.ds(b * CHUNK2, CHUNK2)], a_v.at[p])]

    def g_cps(b, p):
        return [(fs_h.at[src_v.at[p].at[j]],
                 fs_v.at[p].at[pl.ds(j * CB, CB)]) for j in range(SB2)]

    def issue_idx(b, p):
        for s, d in idx_cps(b, p):
            pltpu.async_copy(s, d, semi[p])

    def drain_idx(b, p):
        for s, d in idx_cps(b, p):
            pltpu.make_async_copy(s, d, semi[p]).wait()

    def fire_g(b, p):
        for s, d in g_cps(b, p):
            pltpu.async_copy(s, d, semg[p])

    def drain_g(b, p):
        for s, d in g_cps(b, p):
            pltpu.make_async_copy(s, d, semg[p]).wait()

    def body(b, p):
        def edge(c, _):
            arow = a_v[p, c]
            v0 = jnp.broadcast_to(arow[g], (L,))
            for kq in range(GW // L):
                fs_v[p, c, pl.ds(kq * L, L)] = fs_v[p, c, pl.ds(kq * L, L)] * v0
            return 0

        lax.fori_loop(0, CHUNK2, edge, 0, unroll=2)
        for j in range(SB2):
            pltpu.sync_copy(fs_v.at[p].at[pl.ds(j * CB, CB)],
                            acc_sh.at[dst_v.at[p].at[j]], add=True)

    _db_loop(wid, issue_idx, drain_idx, fire_g, drain_g, body,
             nblk=NBLK2, trips=TRIPS2)
    plsc.subcore_barrier()
    _copy_out_shared(acc_sh, out_h, cid, sid)


def _aggregate(g, fs_g, src, dst, a16):
    k = pl.kernel(
        functools.partial(_agg_body, g),
        out_type=jax.ShapeDtypeStruct((NC, N, GW), jnp.float32),
        mesh=_sc_mesh(),
        compiler_params=pltpu.CompilerParams(use_tc_tiling_on_sc=False),
        scratch_types=[
            pltpu.VMEM((2, SB2, CB), jnp.int32),
            pltpu.VMEM((2, SB2, CB), jnp.int32),
            pltpu.VMEM((2, CHUNK2, L), jnp.float32),
            pltpu.VMEM((2, CHUNK2, GW), jnp.float32),
            pltpu.VMEM((ROWS_LAST // 4, GW), jnp.float32),
            pltpu.VMEM_SHARED((N, GW), jnp.float32),
            pltpu.SemaphoreType.DMA,
            pltpu.SemaphoreType.DMA,
            pltpu.SemaphoreType.DMA,
            pltpu.SemaphoreType.DMA,
        ],
    )
    return k(fs_g, src, dst, a16)


def _combine_body(base_ref, *refs):
    outs = refs[:G]
    out_ref = refs[G]
    b = base_ref[...]
    for g in range(G):
        o = outs[g]
        out_ref[:, g * GW:(g + 1) * GW] = (
            b[:, g * GW:(g + 1) * GW] + o[0] + o[1])


def _combine(rst_base, parts):
    nb = N // ROW_BLK
    pspec = pl.BlockSpec((NC, ROW_BLK, GW), lambda i: (0, i, 0))
    return pl.pallas_call(
        _combine_body,
        grid=(nb,),
        in_specs=[pl.BlockSpec((ROW_BLK, HD), lambda i: (i, 0))] + [pspec] * G,
        out_specs=pl.BlockSpec((ROW_BLK, HD), lambda i: (i, 0)),
        out_shape=jax.ShapeDtypeStruct((N, HD), jnp.float32),
    )(rst_base, *parts)


def kernel(feat, edge_index, e_feat, W_fc, edge_emb, w_r,
           fc_e_W1, fc_e_b1, fc_e_W2, fc_e_b2,
           attn_l, attn_r, attn_e, res_fc_W):
    src = edge_index[0].astype(jnp.int32).reshape(E // CB, CB)
    dst = edge_index[1].astype(jnp.int32).reshape(E // CB, CB)
    et = e_feat.astype(jnp.int32).reshape(E // CB, CB)
    w16 = jnp.broadcast_to(w_r.reshape(H, 1).astype(jnp.float32), (H, L))

    wl, wr, ee16 = _prep(W_fc, attn_l, attn_r, attn_e, edge_emb,
                         fc_e_W1, fc_e_b1, fc_e_W2, fc_e_b2)
    proj = _node_proj(feat, W_fc, res_fc_W, wl, wr)
    fs_slabs = proj[:G]
    rst_base, elp, erp = proj[G:]

    ex16, spart = _edge_logits(src, dst, et, elp, erp, ee16, w16)
    a16 = _normalize(ex16, dst, spart)

    parts = [_aggregate(g, fs_g, src, dst, a16)
             for g, fs_g in enumerate(fs_slabs)]

    rst = _combine(rst_base, parts)
    return rst.reshape(N, H, OUT_F), a16[:, :H]
